# baseline (speedup 1.0000x reference)
import functools

import jax
import jax.numpy as jnp
from jax import lax
from jax.experimental import pallas as pl
from jax.experimental.pallas import tpu as pltpu
from jax.experimental.pallas import tpu_sc as plsc

N = 10000
D = 128
E = 320000
NC = 2
NS = 16
NW = NC * NS
EPW = E // NW
C = 80
NCHUNK = EPW // C
NPT = N // NS
DW = 8
NBUF = 3
DEG_LAG = 12

_mesh = plsc.VectorSubcoreMesh(core_axis_name="c", subcore_axis_name="s")
_sc_params = pltpu.CompilerParams(use_tc_tiling_on_sc=False)


@functools.partial(
    pl.kernel,
    out_type=jax.ShapeDtypeStruct((NC, N, DW), jnp.float32),
    mesh=_mesh,
    scratch_types=[
        pltpu.VMEM((NCHUNK, C), jnp.int32),
        pltpu.VMEM((C, DW), jnp.float32),
        pltpu.VMEM((NPT + 1, DW), jnp.float32),
        pltpu.VMEM_SHARED((N, DW), jnp.float32),
        pltpu.SemaphoreType.DMA,
    ],
    compiler_params=_sc_params,
)
def _deg_kernel(ei_hbm, outp, idxv, onesv, zerov, deg_sh, sem):
    cid = lax.axis_index("c")
    sid = lax.axis_index("s")

    ones2x8 = jnp.ones((2, 8), jnp.float32)
    zeros2x8 = jnp.zeros((2, 8), jnp.float32)

    def fill_ones(i, carry):
        onesv[pl.ds(2 * i, 2)] = ones2x8
        return carry

    lax.fori_loop(0, C // 2, fill_ones, 0)

    def fill_zeros(i, carry):
        zerov[pl.ds(2 * i, 2)] = zeros2x8
        return carry

    lax.fori_loop(0, (NPT + 1) // 2, fill_zeros, 0)

    pltpu.sync_copy(zerov.at[pl.ds(0, NPT)], deg_sh.at[pl.ds(sid * NPT, NPT)])
    plsc.subcore_barrier()

    pltpu.sync_copy(ei_hbm.at[1, cid, sid], idxv)

    def chunk(j, carry):
        pltpu.async_copy(onesv, deg_sh.at[idxv.at[j]], sem, add=True)

        @pl.when(j >= DEG_LAG)
        def _():
            pltpu.make_async_copy(onesv, deg_sh.at[idxv.at[0]], sem).wait()

        return carry

    lax.fori_loop(0, NCHUNK, chunk, 0)
    for _ in range(DEG_LAG):
        pltpu.make_async_copy(onesv, deg_sh.at[idxv.at[0]], sem).wait()
    plsc.subcore_barrier()

    pltpu.sync_copy(deg_sh.at[pl.ds(sid * NPT, NPT)],
                    outp.at[cid, pl.ds(sid * NPT, NPT)])


def _matmul_body(x_ref, w_ref, h_ref):
    h_ref[...] = jnp.dot(x_ref[...], w_ref[...],
                         preferred_element_type=jnp.float32)


def _scale_body(h_ref, degp_ref, g_ref):
    s = degp_ref[0] + degp_ref[1]
    deg = jnp.sum(s, axis=1) * (1.0 / DW) + 1.0
    dinv = lax.rsqrt(deg)
    g_ref[...] = h_ref[...] * dinv[:, None]


@functools.partial(
    pl.kernel,
    out_type=jax.ShapeDtypeStruct((NC, N, D), jnp.float32),
    mesh=_mesh,
    scratch_types=[
        pltpu.VMEM((NCHUNK, C), jnp.int32),
        pltpu.VMEM((NCHUNK, C), jnp.int32),
        pltpu.VMEM((NBUF, C, D), jnp.float32),
        pltpu.VMEM_SHARED((N, D), jnp.float32),
        pltpu.SemaphoreType.DMA,
        pltpu.SemaphoreType.DMA,
    ],
    compiler_params=_sc_params,
)
def _agg_kernel(ei_hbm, g_hbm, zeros_hbm, outp, srcv, dstv, rows,
                agg_sh, sem_g, sem_s):
    cid = lax.axis_index("c")
    sid = lax.axis_index("s")

    @pl.when(cid == 0)
    def _():
        pltpu.sync_copy(g_hbm.at[pl.ds(sid * NPT, NPT)],
                        agg_sh.at[pl.ds(sid * NPT, NPT)])

    @pl.when(cid == 1)
    def _():
        pltpu.sync_copy(zeros_hbm, agg_sh.at[pl.ds(sid * NPT, NPT)])

    plsc.subcore_barrier()

    pltpu.sync_copy(ei_hbm.at[0, cid, sid], srcv)
    pltpu.sync_copy(ei_hbm.at[1, cid, sid], dstv)

    for b in range(NBUF - 1):
        pltpu.async_copy(g_hbm.at[srcv.at[b]], rows.at[b], sem_g)

    def chunk(j, carry):
        buf = rows.at[lax.rem(j, NBUF)]
        pltpu.make_async_copy(g_hbm.at[srcv.at[j]], buf, sem_g).wait()
        pltpu.async_copy(buf, agg_sh.at[dstv.at[j]], sem_s, add=True)

        @pl.when(j >= 1)
        def _():
            pltpu.make_async_copy(rows.at[0], agg_sh.at[dstv.at[j]],
                                  sem_s).wait()

        @pl.when(j + NBUF - 1 < NCHUNK)
        def _():
            nxt = j + NBUF - 1
            pltpu.async_copy(g_hbm.at[srcv.at[nxt]],
                             rows.at[lax.rem(nxt, NBUF)], sem_g)

        return carry

    lax.fori_loop(0, NCHUNK, chunk, 0)
    pltpu.make_async_copy(rows.at[0], agg_sh.at[dstv.at[0]], sem_s).wait()
    plsc.subcore_barrier()

    pltpu.sync_copy(agg_sh.at[pl.ds(sid * NPT, NPT)],
                    outp.at[cid, pl.ds(sid * NPT, NPT)])


def _combine_body(degp_ref, p_ref, b_ref, out_ref):
    s = degp_ref[0] + degp_ref[1]
    deg = jnp.sum(s, axis=1) * (1.0 / DW) + 1.0
    dinv = lax.rsqrt(deg)
    acc = (p_ref[0] + p_ref[1]) * dinv[:, None] + b_ref[0][None, :]
    out_ref[...] = jnp.broadcast_to(acc[None], out_ref.shape)


BN = 2000


def kernel(x, edge_index, W, b):
    ei = edge_index.astype(jnp.int32).reshape(2, NC, NS, NCHUNK, C)

    degp = _deg_kernel(ei)

    h = pl.pallas_call(
        _matmul_body,
        grid=(N // BN,),
        in_specs=[
            pl.BlockSpec((BN, D), lambda i: (i, 0)),
            pl.BlockSpec((D, D), lambda i: (0, 0)),
        ],
        out_specs=pl.BlockSpec((BN, D), lambda i: (i, 0)),
        out_shape=jax.ShapeDtypeStruct((N, D), jnp.float32),
    )(x, W)

    g = pl.pallas_call(
        _scale_body,
        grid=(N // BN,),
        in_specs=[
            pl.BlockSpec((BN, D), lambda i: (i, 0)),
            pl.BlockSpec((NC, BN, DW), lambda i: (0, i, 0)),
        ],
        out_specs=pl.BlockSpec((BN, D), lambda i: (i, 0)),
        out_shape=jax.ShapeDtypeStruct((N, D), jnp.float32),
    )(h, degp)

    partials = _agg_kernel(ei, g, jnp.zeros((NPT, D), jnp.float32))

    out = pl.pallas_call(
        _combine_body,
        grid=(N // BN,),
        in_specs=[
            pl.BlockSpec((NC, BN, DW), lambda i: (0, i, 0)),
            pl.BlockSpec((NC, BN, D), lambda i: (0, i, 0)),
            pl.BlockSpec((1, D), lambda i: (0, 0)),
        ],
        out_specs=pl.BlockSpec((5, BN, D), lambda i: (0, i, 0)),
        out_shape=jax.ShapeDtypeStruct((5, N, D), jnp.float32),
    )(degp, partials, b.reshape(1, D))

    return out


# speedup vs baseline: 1.0505x; 1.0505x over previous
import functools

import jax
import jax.numpy as jnp
from jax import lax
from jax.experimental import pallas as pl
from jax.experimental.pallas import tpu as pltpu
from jax.experimental.pallas import tpu_sc as plsc

N = 10000
D = 128
E = 320000
NC = 2
NS = 16
NW = NC * NS
EPW = E // NW
C = 80
NCHUNK = EPW // C
NPT = N // NS
DW = 8
NBUF = 4
NIX = NBUF + 2
DEG_LAG = 12

_mesh = plsc.VectorSubcoreMesh(core_axis_name="c", subcore_axis_name="s")
_sc_params = pltpu.CompilerParams(use_tc_tiling_on_sc=False)


@functools.partial(
    pl.kernel,
    out_type=jax.ShapeDtypeStruct((NC, N, DW), jnp.float32),
    mesh=_mesh,
    scratch_types=[
        pltpu.VMEM((NCHUNK, C), jnp.int32),
        pltpu.VMEM((C, DW), jnp.float32),
        pltpu.VMEM((NPT + 1, DW), jnp.float32),
        pltpu.VMEM_SHARED((N, DW), jnp.float32),
        pltpu.SemaphoreType.DMA,
    ],
    compiler_params=_sc_params,
)
def _deg_kernel(ei_hbm, outp, idxv, onesv, zerov, deg_sh, sem):
    cid = lax.axis_index("c")
    sid = lax.axis_index("s")

    ones2x8 = jnp.ones((2, 8), jnp.float32)
    zeros2x8 = jnp.zeros((2, 8), jnp.float32)

    def fill_ones(i, carry):
        onesv[pl.ds(2 * i, 2)] = ones2x8
        return carry

    lax.fori_loop(0, C // 2, fill_ones, 0)

    def fill_zeros(i, carry):
        zerov[pl.ds(2 * i, 2)] = zeros2x8
        return carry

    lax.fori_loop(0, (NPT + 1) // 2, fill_zeros, 0)

    pltpu.sync_copy(zerov.at[pl.ds(0, NPT)], deg_sh.at[pl.ds(sid * NPT, NPT)])
    plsc.subcore_barrier()

    pltpu.sync_copy(ei_hbm.at[1, cid, sid], idxv)

    def chunk(j, carry):
        pltpu.async_copy(onesv, deg_sh.at[idxv.at[j]], sem, add=True)

        @pl.when(j >= DEG_LAG)
        def _():
            pltpu.make_async_copy(onesv, deg_sh.at[idxv.at[0]], sem).wait()

        return carry

    lax.fori_loop(0, NCHUNK, chunk, 0)
    for _ in range(DEG_LAG):
        pltpu.make_async_copy(onesv, deg_sh.at[idxv.at[0]], sem).wait()
    plsc.subcore_barrier()

    pltpu.sync_copy(deg_sh.at[pl.ds(sid * NPT, NPT)],
                    outp.at[cid, pl.ds(sid * NPT, NPT)])


def _matmul_body(x_ref, w_ref, h_ref):
    h_ref[...] = jnp.dot(x_ref[...], w_ref[...],
                         preferred_element_type=jnp.float32)


def _scale_body(h_ref, degp_ref, g_ref):
    s = degp_ref[0] + degp_ref[1]
    deg = jnp.sum(s, axis=1) * (1.0 / DW) + 1.0
    dinv = lax.rsqrt(deg)
    g_ref[...] = h_ref[...] * dinv[:, None]


@functools.partial(
    pl.kernel,
    out_type=jax.ShapeDtypeStruct((NC, N, D), jnp.float32),
    mesh=_mesh,
    scratch_types=[
        pltpu.VMEM((NIX, C), jnp.int32),
        pltpu.VMEM((NIX, C), jnp.int32),
        pltpu.VMEM((NBUF, C, D), jnp.float32),
        pltpu.VMEM_SHARED((N, D), jnp.float32),
        pltpu.SemaphoreType.DMA,
        pltpu.SemaphoreType.DMA,
        pltpu.SemaphoreType.DMA,
    ],
    compiler_params=_sc_params,
)
def _agg_kernel(ei_hbm, g_hbm, zeros_hbm, outp, srcv, dstv, rows,
                agg_sh, sem_g, sem_s, sem_i):
    cid = lax.axis_index("c")
    sid = lax.axis_index("s")

    @pl.when(cid == 0)
    def _():
        pltpu.sync_copy(g_hbm.at[pl.ds(sid * NPT, NPT)],
                        agg_sh.at[pl.ds(sid * NPT, NPT)])

    @pl.when(cid == 1)
    def _():
        pltpu.sync_copy(zeros_hbm, agg_sh.at[pl.ds(sid * NPT, NPT)])

    plsc.subcore_barrier()

    for k in range(NIX):
        pltpu.async_copy(ei_hbm.at[0, cid, sid].at[k], srcv.at[k], sem_i)
        pltpu.async_copy(ei_hbm.at[1, cid, sid].at[k], dstv.at[k], sem_i)
    for _ in range(2 * (NBUF - 1)):
        pltpu.make_async_copy(ei_hbm.at[0, cid, sid].at[0], srcv.at[0],
                              sem_i).wait()
    for b in range(NBUF - 1):
        pltpu.async_copy(g_hbm.at[srcv.at[b]], rows.at[b], sem_g)

    def chunk(j, carry):
        buf = rows.at[lax.rem(j, NBUF)]
        pltpu.make_async_copy(g_hbm.at[srcv.at[0]], buf, sem_g).wait()
        pltpu.async_copy(buf, agg_sh.at[dstv.at[lax.rem(j, NIX)]], sem_s,
                         add=True)

        @pl.when(j >= 1)
        def _():
            pltpu.make_async_copy(rows.at[0], agg_sh.at[dstv.at[0]],
                                  sem_s).wait()

        @pl.when(j + NBUF - 1 < NCHUNK)
        def _():
            nxt = j + NBUF - 1
            pltpu.make_async_copy(ei_hbm.at[0, cid, sid].at[0], srcv.at[0],
                                  sem_i).wait()
            pltpu.make_async_copy(ei_hbm.at[0, cid, sid].at[0], srcv.at[0],
                                  sem_i).wait()
            pltpu.async_copy(g_hbm.at[srcv.at[lax.rem(nxt, NIX)]],
                             rows.at[lax.rem(nxt, NBUF)], sem_g)

        @pl.when(jnp.logical_and(j >= 1, j + NIX - 1 < NCHUNK))
        def _():
            nix = j + NIX - 1
            slot = lax.rem(nix, NIX)
            pltpu.async_copy(ei_hbm.at[0, cid, sid].at[nix], srcv.at[slot],
                             sem_i)
            pltpu.async_copy(ei_hbm.at[1, cid, sid].at[nix], dstv.at[slot],
                             sem_i)

        return carry

    lax.fori_loop(0, NCHUNK, chunk, 0)
    pltpu.make_async_copy(rows.at[0], agg_sh.at[dstv.at[0]], sem_s).wait()
    plsc.subcore_barrier()

    pltpu.sync_copy(agg_sh.at[pl.ds(sid * NPT, NPT)],
                    outp.at[cid, pl.ds(sid * NPT, NPT)])


def _combine_body(degp_ref, p_ref, b_ref, out_ref):
    s = degp_ref[0] + degp_ref[1]
    deg = jnp.sum(s, axis=1) * (1.0 / DW) + 1.0
    dinv = lax.rsqrt(deg)
    acc = (p_ref[0] + p_ref[1]) * dinv[:, None] + b_ref[0][None, :]
    out_ref[...] = jnp.broadcast_to(acc[None], out_ref.shape)


BN = 2000


def kernel(x, edge_index, W, b):
    ei = edge_index.astype(jnp.int32).reshape(2, NC, NS, NCHUNK, C)

    degp = _deg_kernel(ei)

    h = pl.pallas_call(
        _matmul_body,
        grid=(N // BN,),
        in_specs=[
            pl.BlockSpec((BN, D), lambda i: (i, 0)),
            pl.BlockSpec((D, D), lambda i: (0, 0)),
        ],
        out_specs=pl.BlockSpec((BN, D), lambda i: (i, 0)),
        out_shape=jax.ShapeDtypeStruct((N, D), jnp.float32),
    )(x, W)

    g = pl.pallas_call(
        _scale_body,
        grid=(N // BN,),
        in_specs=[
            pl.BlockSpec((BN, D), lambda i: (i, 0)),
            pl.BlockSpec((NC, BN, DW), lambda i: (0, i, 0)),
        ],
        out_specs=pl.BlockSpec((BN, D), lambda i: (i, 0)),
        out_shape=jax.ShapeDtypeStruct((N, D), jnp.float32),
    )(h, degp)

    partials = _agg_kernel(ei, g, jnp.zeros((NPT, D), jnp.float32))

    out = pl.pallas_call(
        _combine_body,
        grid=(N // BN,),
        in_specs=[
            pl.BlockSpec((NC, BN, DW), lambda i: (0, i, 0)),
            pl.BlockSpec((NC, BN, D), lambda i: (0, i, 0)),
            pl.BlockSpec((1, D), lambda i: (0, 0)),
        ],
        out_specs=pl.BlockSpec((5, BN, D), lambda i: (0, i, 0)),
        out_shape=jax.ShapeDtypeStruct((5, N, D), jnp.float32),
    )(degp, partials, b.reshape(1, D))

    return out
